# Initial kernel scaffold; baseline (speedup 1.0000x reference)
import jax
import jax.numpy as jnp
from jax.experimental import pallas as pl


def kernel(indices, table):
    raise NotImplementedError("write your pallas kernel here")


# speedup vs baseline: 3.6646x; 3.6646x over previous
import jax
import jax.numpy as jnp
from jax.experimental import pallas as pl
from jax.experimental.pallas import tpu as pltpu


def _round_up(x, m):
    return ((x + m - 1) // m) * m


def _make_gather_kernel(rows, unroll):
    n_chunks = rows // unroll

    def _kernel(idx_ref, table_ref, out_ref):
        def chunk(c, carry):
            base = c * unroll
            for u in range(unroll):
                i = idx_ref[0, 0, base + u]
                out_ref[base + u] = table_ref[i]
            return carry

        jax.lax.fori_loop(0, n_chunks, chunk, 0, unroll=False)

    return _kernel


def _embedding_gather(indices, table, *, rows_per_block=2048, unroll=64):
    B, F = indices.shape
    V, D = table.shape
    n = B * F

    flat_idx = jnp.clip(indices.reshape(-1).astype(jnp.int32), 0, V - 1)

    D_pad = _round_up(D, 128)
    table_p = table if D_pad == D else jnp.pad(table, ((0, 0), (0, D_pad - D)))
    table3 = table_p.reshape(V, 1, D_pad)

    R = rows_per_block
    if n < R:
        R = max(unroll, _round_up(n, unroll))
    n_pad = _round_up(n, R)
    if n_pad != n:
        flat_idx = jnp.pad(flat_idx, (0, n_pad - n))
    n_blocks = n_pad // R
    idx3 = flat_idx.reshape(n_blocks, 1, R)

    itemsize = jnp.dtype(table_p.dtype).itemsize
    table_bytes = V * D_pad * itemsize
    out_block_bytes = R * D_pad * itemsize
    vmem_limit = int(min(2 * table_bytes + 4 * out_block_bytes + (8 << 20), 120 << 20))

    cost = pl.CostEstimate(
        flops=0,
        transcendentals=0,
        bytes_accessed=table_bytes + n_pad * 4 + n_pad * D_pad * itemsize,
    )

    out = pl.pallas_call(
        _make_gather_kernel(R, unroll),
        out_shape=jax.ShapeDtypeStruct((n_pad, 1, D_pad), table_p.dtype),
        grid=(n_blocks,),
        in_specs=[
            pl.BlockSpec((1, 1, R), lambda i: (i, 0, 0), memory_space=pltpu.SMEM),
            pl.BlockSpec((V, 1, D_pad), lambda i: (0, 0, 0)),
        ],
        out_specs=pl.BlockSpec((R, 1, D_pad), lambda i: (i, 0, 0)),
        compiler_params=pltpu.CompilerParams(
            dimension_semantics=("parallel",),
            vmem_limit_bytes=vmem_limit,
        ),
        cost_estimate=cost,
    )(idx3, table3)

    return out[:n, 0, :D].reshape(B, F, D)


def kernel(indices, table):
    return _embedding_gather(indices, table)


# speedup vs baseline: 5.8028x; 1.5835x over previous
import jax
import jax.numpy as jnp
from jax.experimental import pallas as pl
from jax.experimental.pallas import tpu as pltpu


def _make_gather_kernel(g_rows, num_fields):
    def _kernel(idx_ref, table_ref, out_ref):
        def per_row(g, carry):
            pos = g * num_fields
            for f in range(num_fields):
                v = idx_ref[0, 0, pos + f]
                base = pl.multiple_of((v >> 3) << 3, 8)
                chunk = table_ref[pl.ds(base, 8), :]
                rolled = pltpu.roll(chunk, v & 7, axis=0)
                row = rolled[(f % 8):(f % 8) + 1, :]
                out_ref[pl.ds(g, 1), pl.ds(f, 1), :] = row[None]
            return carry

        jax.lax.fori_loop(0, g_rows, per_row, 0, unroll=False)

    return _kernel


def _embedding_gather(indices, table, *, batch_block=64):
    B, F = indices.shape
    V, D = table.shape

    idx = jnp.clip(indices.astype(jnp.int32), 0, V - 1)
    f_iota = jax.lax.broadcasted_iota(jnp.int32, (B, F), 1)
    packed = (idx & ~7) | ((f_iota - idx) & 7)

    G = batch_block
    n_blocks = B // G
    idx3 = packed.reshape(n_blocks, 1, G * F)

    table_bytes = V * D * 4
    out_block_bytes = G * F * D * 4
    vmem_limit = int(min(2 * table_bytes + 4 * out_block_bytes + (8 << 20), 120 << 20))

    cost = pl.CostEstimate(
        flops=0,
        transcendentals=0,
        bytes_accessed=table_bytes + B * F * 4 + B * F * D * 4,
    )

    return pl.pallas_call(
        _make_gather_kernel(G, F),
        out_shape=jax.ShapeDtypeStruct((B, F, D), table.dtype),
        grid=(n_blocks,),
        in_specs=[
            pl.BlockSpec((1, 1, G * F), lambda i: (i, 0, 0), memory_space=pltpu.SMEM),
            pl.BlockSpec((V, D), lambda i: (0, 0)),
        ],
        out_specs=pl.BlockSpec((G, F, D), lambda i: (i, 0, 0)),
        compiler_params=pltpu.CompilerParams(
            dimension_semantics=("parallel",),
            vmem_limit_bytes=vmem_limit,
        ),
        cost_estimate=cost,
    )(idx3, table)


def kernel(indices, table):
    return _embedding_gather(indices, table)


# speedup vs baseline: 9.2035x; 1.5860x over previous
import jax
import jax.numpy as jnp
import numpy as np
from jax.experimental import pallas as pl
from jax.experimental.pallas import tpu as pltpu
from jax.experimental.shard_map import shard_map
from jax.sharding import Mesh, PartitionSpec as P


def _make_gather_kernel(g_rows, num_fields):
    n_tiles = (num_fields + 7) // 8

    def _kernel(idx_ref, table_ref, out_ref):
        d = table_ref.shape[1]
        sub_iota = jax.lax.broadcasted_iota(jnp.int32, (8, d), 0)

        def per_row(g, carry):
            pos = g * num_fields
            tiles = [None] * n_tiles
            for f in range(num_fields):
                v = idx_ref[0, 0, pos + f]
                base = pl.multiple_of((v >> 3) << 3, 8)
                chunk = table_ref[pl.ds(base, 8), :]
                rolled = pltpu.roll(chunk, v & 7, axis=0)
                t = f // 8
                if tiles[t] is None:
                    tiles[t] = rolled
                else:
                    tiles[t] = jnp.where(sub_iota == (f % 8), rolled, tiles[t])
            for t in range(n_tiles):
                hi = min((t + 1) * 8, num_fields)
                out_ref[pl.ds(g, 1), t * 8:hi, :] = tiles[t][None, : hi - t * 8, :]
            return carry

        jax.lax.fori_loop(0, g_rows, per_row, 0, unroll=False)

    return _kernel


def _embedding_gather(indices, table, *, batch_block=64):
    B, F = indices.shape
    V, D = table.shape

    idx = jnp.clip(indices.astype(jnp.int32), 0, V - 1)
    f_iota = jax.lax.broadcasted_iota(jnp.int32, (B, F), 1)
    packed = (idx & ~7) | ((f_iota - idx) & 7)

    G = batch_block
    n_blocks = B // G
    idx3 = packed.reshape(n_blocks, 1, G * F)

    table_bytes = V * D * 4
    out_block_bytes = G * F * D * 4
    vmem_limit = int(min(2 * table_bytes + 4 * out_block_bytes + (8 << 20), 120 << 20))

    cost = pl.CostEstimate(
        flops=0,
        transcendentals=0,
        bytes_accessed=table_bytes + B * F * 4 + B * F * D * 4,
    )

    return pl.pallas_call(
        _make_gather_kernel(G, F),
        out_shape=jax.ShapeDtypeStruct((B, F, D), table.dtype),
        grid=(n_blocks,),
        in_specs=[
            pl.BlockSpec((1, 1, G * F), lambda i: (i, 0, 0), memory_space=pltpu.SMEM),
            pl.BlockSpec((V, D), lambda i: (0, 0)),
        ],
        out_specs=pl.BlockSpec((G, F, D), lambda i: (i, 0, 0)),
        compiler_params=pltpu.CompilerParams(
            dimension_semantics=("parallel",),
            vmem_limit_bytes=vmem_limit,
        ),
        cost_estimate=cost,
    )(idx3, table)


def kernel(indices, table):
    B = indices.shape[0]
    devs = jax.devices()
    n_dev = 2 if (len(devs) >= 2 and B % 128 == 0) else 1
    if n_dev == 1:
        return _embedding_gather(indices, table)
    mesh = Mesh(np.array(devs[:2]), ("x",))
    fn = shard_map(
        _embedding_gather,
        mesh=mesh,
        in_specs=(P("x", None), P(None, None)),
        out_specs=P("x", None, None),
        check_rep=False,
    )
    return fn(indices, table)
